# Initial kernel scaffold; baseline (speedup 1.0000x reference)
import jax
import jax.numpy as jnp
from jax.experimental import pallas as pl


def kernel(output, target, cls_weights, myLambda, embed):
    raise NotImplementedError("write your pallas kernel here")


# speedup vs baseline: 1.7039x; 1.7039x over previous
import jax
import jax.numpy as jnp
from jax.experimental import pallas as pl

_B = 16384
_C = 1000
_R = 512
_NB = _B // _R


def _ce_body(x_ref, t_ref, w_ref, out_ref):
    x = x_ref[...]
    t = t_ref[0]
    m = jnp.max(x, axis=1, keepdims=True)
    s = jnp.sum(jnp.exp(x - m), axis=1, keepdims=True)
    lse = m + jnp.log(s)
    cols = jax.lax.broadcasted_iota(jnp.int32, (_R, _C), 1)
    onehot = cols == t
    xt = jnp.sum(jnp.where(onehot, x, 0.0), axis=1, keepdims=True)
    wt = jnp.sum(jnp.where(onehot, w_ref[...], 0.0), axis=1, keepdims=True)
    partial = jnp.sum(wt * (lse - xt), keepdims=True) * (1.0 / _B)

    @pl.when(pl.program_id(0) == 0)
    def _():
        out_ref[...] = jnp.zeros_like(out_ref)

    out_ref[...] += partial


def kernel(output, target, cls_weights, myLambda, embed):
    t3 = target.astype(jnp.int32).reshape(_NB, _R, 1)
    w2 = cls_weights.reshape(1, _C)
    out = pl.pallas_call(
        _ce_body,
        grid=(_NB,),
        in_specs=[
            pl.BlockSpec((_R, _C), lambda i: (i, 0)),
            pl.BlockSpec((1, _R, 1), lambda i: (i, 0, 0)),
            pl.BlockSpec((1, _C), lambda i: (0, 0)),
        ],
        out_specs=pl.BlockSpec((1, 1), lambda i: (0, 0)),
        out_shape=jax.ShapeDtypeStruct((1, 1), jnp.float32),
    )(output, t3, w2)
    return out[0, 0]


# speedup vs baseline: 1.8911x; 1.1099x over previous
import jax
import jax.numpy as jnp
from jax.experimental import pallas as pl

_B = 16384
_C = 1000
_R = 1024
_NB = _B // _R


def _ce_body(x_ref, t_ref, w_ref, out_ref):
    x = x_ref[...]
    t = t_ref[0]
    m = jnp.max(x, axis=1, keepdims=True)
    s = jnp.sum(jnp.exp(x - m), axis=1, keepdims=True)
    lse = m + jnp.log(s)
    cols = jax.lax.broadcasted_iota(jnp.int32, (_R, _C), 1)
    onehot = cols == t
    xt = jnp.sum(jnp.where(onehot, x, 0.0), axis=1, keepdims=True)
    wt = jnp.sum(jnp.where(onehot, w_ref[...], 0.0), axis=1, keepdims=True)
    partial = jnp.sum(wt * (lse - xt), keepdims=True) * (1.0 / _B)

    @pl.when(pl.program_id(0) == 0)
    def _():
        out_ref[...] = jnp.zeros_like(out_ref)

    out_ref[...] += partial


def kernel(output, target, cls_weights, myLambda, embed):
    t3 = target.astype(jnp.int32).reshape(_NB, _R, 1)
    w2 = cls_weights.reshape(1, _C)
    out = pl.pallas_call(
        _ce_body,
        grid=(_NB,),
        in_specs=[
            pl.BlockSpec((_R, _C), lambda i: (i, 0)),
            pl.BlockSpec((1, _R, 1), lambda i: (i, 0, 0)),
            pl.BlockSpec((1, _C), lambda i: (0, 0)),
        ],
        out_specs=pl.BlockSpec((1, 1), lambda i: (0, 0)),
        out_shape=jax.ShapeDtypeStruct((1, 1), jnp.float32),
    )(output, t3, w2)
    return out[0, 0]
